# baseline (speedup 1.0000x reference)
import functools

import jax
import jax.numpy as jnp
from jax import lax
from jax.experimental import pallas as pl
from jax.experimental.pallas import tpu as pltpu
from jax.experimental.pallas import tpu_sc as plsc

B, P, C_IN = 4, 8192, 16
M = P // 4
K = 32
C1, C2 = 32, 64
BM = 64
PB = P // (M // BM)
RB = 256
NW = 32
ROWS = B * M * K
CH = 128
EPS = 1e-5


NCH = 64
SCH = 128
DL = 6


def _k1_body(xyzt_ref, xyz_ref, featsT_ref, cen_ref, w1t_ref, b1_ref,
             idx_ref, cproj_ref, ptab_ref):
    xt4 = xyzt_ref[0]
    cen = cen_ref[0]
    pn = jnp.sum(xt4 * xt4, axis=0)
    cn = jnp.sum(cen * cen, axis=1)
    cb = cen.astype(jnp.bfloat16).astype(jnp.float32)
    xb = xt4.astype(jnp.bfloat16).astype(jnp.float32)
    dot = (cb[:, 0][:, None, None] * xb[0][None]
           + cb[:, 1][:, None, None] * xb[1][None]
           + cb[:, 2][:, None, None] * xb[2][None])
    d0 = cn[:, None, None] + pn[None] - 2.0 * dot

    iota_l = lax.broadcasted_iota(jnp.int32, (BM, NCH, SCH), 2)
    iota_c = lax.broadcasted_iota(jnp.int32, (BM, NCH, SCH), 1)
    gi4 = iota_c * SCH + iota_l
    ch_iota = lax.broadcasted_iota(jnp.int32, (BM, NCH), 1)
    liota = lax.broadcasted_iota(jnp.int32, (BM, DL, NCH), 1)
    kiota = lax.broadcasted_iota(jnp.int32, (BM, K), 1)

    def lev(l, carry):
        dw, v, i = carry
        m = jnp.min(dw, axis=2)
        sel = jnp.where(dw == m[:, :, None], iota_l, SCH)
        a = jnp.min(sel, axis=2)
        dw = jnp.where(iota_l == a[:, :, None], jnp.inf, dw)
        gi = a + SCH * ch_iota
        v = jnp.where(liota == l, m[:, None, :], v)
        i = jnp.where(liota == l, gi[:, None, :], i)
        return dw, v, i

    _, v, i = lax.fori_loop(0, DL, lev, (
        d0,
        jnp.full((BM, DL, NCH), jnp.inf, jnp.float32),
        jnp.zeros((BM, DL, NCH), jnp.int32)))

    def step(k, carry):
        v, acc, lastm, lasti = carry
        m = jnp.min(jnp.min(v, axis=2), axis=1)
        m3 = m[:, None, None]
        cand = jnp.where(v == m3, i, P)
        ii = jnp.min(jnp.min(cand, axis=2), axis=1)
        ii3 = ii[:, None, None]
        acc = jnp.where(kiota == k, ii[:, None], acc)
        v = jnp.where((v == m3) & (i == ii3), jnp.inf, v)
        return v, acc, m[:, None], ii[:, None]

    _, acc, lastm, lasti = lax.fori_loop(0, K, step, (
        v, jnp.zeros((BM, K), jnp.int32),
        jnp.zeros((BM, 1), jnp.float32), jnp.zeros((BM, 1), jnp.int32)))

    lm3 = lastm[:, :, None]
    li3 = lasti[:, :, None]
    lex = (d0 < lm3) | ((d0 == lm3) & (gi4 < li3))
    cnt = jnp.sum(jnp.sum(lex.astype(jnp.int32), axis=2), axis=1)
    bad = jnp.any(cnt != K - 1)

    def _naive():
        def nstep(k, carry):
            dd, acc2 = carry
            m = jnp.min(jnp.min(dd, axis=2), axis=1)[:, None, None]
            ci = jnp.min(jnp.min(jnp.where(dd == m, gi4, P), axis=2), axis=1)
            acc2 = jnp.where(kiota == k, ci[:, None], acc2)
            dd = jnp.where(gi4 == ci[:, None, None], jnp.inf, dd)
            return dd, acc2

        return lax.fori_loop(0, K, nstep,
                             (d0, jnp.zeros((BM, K), jnp.int32)))[1]

    acc = lax.cond(bad, _naive, lambda: acc)
    b = pl.program_id(0)
    idx_ref[0] = acc + b * P

    w1t = w1t_ref[...]
    cproj = (cen[:, 0:1] * w1t[0:1, :]
             + cen[:, 1:2] * w1t[1:2, :]
             + cen[:, 2:3] * w1t[2:3, :]) - b1_ref[...]
    cproj_ref[0] = cproj

    xb = xyz_ref[0]
    fb = featsT_ref[0]
    pt = (xb[:, 0:1] * w1t[0:1, :]
          + xb[:, 1:2] * w1t[1:2, :]
          + xb[:, 2:3] * w1t[2:3, :])
    pt = pt + jnp.dot(fb, w1t[3:, :], preferred_element_type=jnp.float32)
    ptab_ref[0] = pt


def _knn_project(xyz_t, xyz, featsT, centers, w1t, b1r):
    return pl.pallas_call(
        _k1_body,
        grid=(B, M // BM),
        in_specs=[
            pl.BlockSpec((1, 3, NCH, SCH), lambda b, i: (b, 0, 0, 0)),
            pl.BlockSpec((1, PB, 3), lambda b, i: (b, i, 0)),
            pl.BlockSpec((1, PB, C_IN), lambda b, i: (b, i, 0)),
            pl.BlockSpec((1, BM, 3), lambda b, i: (b, i, 0)),
            pl.BlockSpec((3 + C_IN, C1), lambda b, i: (0, 0)),
            pl.BlockSpec((1, C1), lambda b, i: (0, 0)),
        ],
        out_specs=[
            pl.BlockSpec((1, BM, K), lambda b, i: (b, i, 0)),
            pl.BlockSpec((1, BM, C1), lambda b, i: (b, i, 0)),
            pl.BlockSpec((1, PB, C1), lambda b, i: (b, i, 0)),
        ],
        out_shape=[
            jax.ShapeDtypeStruct((B, M, K), jnp.int32),
            jax.ShapeDtypeStruct((B, M, C1), jnp.float32),
            jax.ShapeDtypeStruct((B, P, C1), jnp.float32),
        ],
    )(xyz_t, xyz, featsT, centers, w1t, b1r)


@functools.partial(
    pl.kernel,
    mesh=plsc.VectorSubcoreMesh(core_axis_name="c", subcore_axis_name="s"),
    compiler_params=pltpu.CompilerParams(use_tc_tiling_on_sc=False),
    out_type=jax.ShapeDtypeStruct((ROWS, C1), jnp.float32),
    scratch_types=[
        pltpu.VMEM((CH,), jnp.int32),
        pltpu.VMEM((CH, C1), jnp.float32),
        pltpu.SemaphoreType.DMA,
    ],
)
def _sc_gather(table_hbm, idx_hbm, out_hbm, idx_v, rows_v, sem):
    wid = lax.axis_index("s") * 2 + lax.axis_index("c")
    per_w = ROWS // NW
    base = wid * per_w

    def body(c, carry):
        off = base + c * CH
        pltpu.sync_copy(idx_hbm.at[pl.ds(off, CH)], idx_v)
        pltpu.async_copy(table_hbm.at[idx_v], rows_v, sem).wait()
        pltpu.sync_copy(rows_v, out_hbm.at[pl.ds(off, CH)])
        return carry

    lax.fori_loop(0, per_w // CH, body, 0)


def _k3_body(g_ref, cp_ref, sums_ref):
    h1 = g_ref[...] - cp_ref[...][:, None, :]
    s1 = jnp.sum(jnp.sum(h1, axis=0), axis=0)
    s2 = jnp.sum(jnp.sum(h1 * h1, axis=0), axis=0)

    @pl.when(pl.program_id(0) == 0)
    def _():
        sums_ref[...] = jnp.zeros_like(sums_ref)

    sums_ref[0:1, :] += s1[None, :]
    sums_ref[1:2, :] += s2[None, :]


def _k4_body(g_ref, cp_ref, a1_ref, s1_ref, w2t_ref, b2_ref, sums_ref):
    h1 = g_ref[...] - cp_ref[...][:, None, :]
    x1 = jnp.maximum(h1 * a1_ref[0][None, None, :]
                     + s1_ref[0][None, None, :], 0.0)
    x1f = x1.reshape(RB * K, C1)
    h2 = jnp.dot(x1f, w2t_ref[...], preferred_element_type=jnp.float32)
    h2 = h2 + b2_ref[...]
    s1 = jnp.sum(h2, axis=0)
    s2 = jnp.sum(h2 * h2, axis=0)

    @pl.when(pl.program_id(0) == 0)
    def _():
        sums_ref[...] = jnp.zeros_like(sums_ref)

    sums_ref[0:1, :] += s1[None, :]
    sums_ref[1:2, :] += s2[None, :]


def _k5_body(g_ref, cp_ref, a1_ref, s1_ref, w2t_ref, b2_ref, a2_ref, s2_ref,
             out_ref):
    h1 = g_ref[...] - cp_ref[...][:, None, :]
    x1 = jnp.maximum(h1 * a1_ref[0][None, None, :]
                     + s1_ref[0][None, None, :], 0.0)
    x1f = x1.reshape(RB * K, C1)
    h2 = jnp.dot(x1f, w2t_ref[...], preferred_element_type=jnp.float32)
    h2 = h2 + b2_ref[...]
    x2 = jnp.maximum(h2 * a2_ref[...] + s2_ref[...], 0.0)
    x3 = x2.reshape(RB, K, C2)
    mx = x3[:, 0, :]
    for k in range(1, K):
        mx = jnp.maximum(mx, x3[:, k, :])
    out_ref[...] = mx


def kernel(xyz, feats, W1, b1, g1, be1, W2, b2, g2, be2):
    idxc = jnp.linspace(0.0, P - 1, M).astype(jnp.int32)
    centers = jnp.take(xyz, idxc, axis=1)

    xyz_t = xyz.transpose(0, 2, 1).reshape(B, 3, NCH, SCH)
    featsT = feats.transpose(0, 2, 1)
    w1t = W1.T
    b1r = b1.reshape(1, C1)

    idx, cproj, ptable = _knn_project(xyz_t, xyz, featsT, centers, w1t, b1r)

    g = _sc_gather(ptable.reshape(B * P, C1), idx.reshape(ROWS))
    g3 = g.reshape(B * M, K, C1)
    cpf = cproj.reshape(B * M, C1)

    nblk = (B * M) // RB
    sums1 = pl.pallas_call(
        _k3_body,
        grid=(nblk,),
        in_specs=[
            pl.BlockSpec((RB, K, C1), lambda i: (i, 0, 0)),
            pl.BlockSpec((RB, C1), lambda i: (i, 0)),
        ],
        out_specs=pl.BlockSpec((8, C1), lambda i: (0, 0)),
        out_shape=jax.ShapeDtypeStruct((8, C1), jnp.float32),
    )(g3, cpf)

    n1 = float(ROWS)
    mean1 = sums1[0] / n1
    var1 = sums1[1] / n1 - mean1 * mean1
    sc1 = g1 / jnp.sqrt(var1 + EPS)
    sh1 = be1 - mean1 * sc1
    w2t = W2.T
    b2r = b2.reshape(1, C2)

    sums2 = pl.pallas_call(
        _k4_body,
        grid=(nblk,),
        in_specs=[
            pl.BlockSpec((RB, K, C1), lambda i: (i, 0, 0)),
            pl.BlockSpec((RB, C1), lambda i: (i, 0)),
            pl.BlockSpec((1, C1), lambda i: (0, 0)),
            pl.BlockSpec((1, C1), lambda i: (0, 0)),
            pl.BlockSpec((C1, C2), lambda i: (0, 0)),
            pl.BlockSpec((1, C2), lambda i: (0, 0)),
        ],
        out_specs=pl.BlockSpec((8, C2), lambda i: (0, 0)),
        out_shape=jax.ShapeDtypeStruct((8, C2), jnp.float32),
    )(g3, cpf, sc1.reshape(1, C1), sh1.reshape(1, C1), w2t, b2r)

    mean2 = sums2[0] / n1
    var2 = sums2[1] / n1 - mean2 * mean2
    sc2 = g2 / jnp.sqrt(var2 + EPS)
    sh2 = be2 - mean2 * sc2

    out2 = pl.pallas_call(
        _k5_body,
        grid=(nblk,),
        in_specs=[
            pl.BlockSpec((RB, K, C1), lambda i: (i, 0, 0)),
            pl.BlockSpec((RB, C1), lambda i: (i, 0)),
            pl.BlockSpec((1, C1), lambda i: (0, 0)),
            pl.BlockSpec((1, C1), lambda i: (0, 0)),
            pl.BlockSpec((C1, C2), lambda i: (0, 0)),
            pl.BlockSpec((1, C2), lambda i: (0, 0)),
            pl.BlockSpec((1, C2), lambda i: (0, 0)),
            pl.BlockSpec((1, C2), lambda i: (0, 0)),
        ],
        out_specs=pl.BlockSpec((RB, C2), lambda i: (i, 0)),
        out_shape=jax.ShapeDtypeStruct((B * M, C2), jnp.float32),
    )(g3, cpf, sc1.reshape(1, C1), sh1.reshape(1, C1), w2t, b2r,
      sc2.reshape(1, C2), sh2.reshape(1, C2))

    out = out2.reshape(B, M, C2).transpose(0, 2, 1)
    return centers, out


# speedup vs baseline: 2.1674x; 2.1674x over previous
import functools

import jax
import jax.numpy as jnp
from jax import lax
from jax.experimental import pallas as pl
from jax.experimental.pallas import tpu as pltpu
from jax.experimental.pallas import tpu_sc as plsc

B, P, C_IN = 4, 8192, 16
M = P // 4
K = 32
C1, C2 = 32, 64
BM = 64
PB = P // (M // BM)
RB = 256
NW = 32
ROWS = B * M * K
CH = 128
EPS = 1e-5


NCH = 64
SCH = 128
DL = 6


def _k1_body(xyzt_ref, xyz_ref, featsT_ref, cen_ref, w1t_ref, b1_ref,
             idx_ref, cproj_ref, ptab_ref):
    xt4 = xyzt_ref[0]
    cen = cen_ref[0]
    pn = jnp.sum(xt4 * xt4, axis=0)
    cn = jnp.sum(cen * cen, axis=1)
    cb = cen.astype(jnp.bfloat16).astype(jnp.float32)
    xb = xt4.astype(jnp.bfloat16).astype(jnp.float32)
    dot = (cb[:, 0][None, :, None] * xb[0][:, None, :]
           + cb[:, 1][None, :, None] * xb[1][:, None, :]
           + cb[:, 2][None, :, None] * xb[2][:, None, :])
    d0 = cn[None, :, None] + pn[:, None, :] - 2.0 * dot

    iota_g = lax.broadcasted_iota(jnp.int32, (NCH, BM, SCH), 0)
    iota_l = lax.broadcasted_iota(jnp.int32, (NCH, BM, SCH), 2)
    gi4 = iota_g * SCH + iota_l
    lane_iota = lax.broadcasted_iota(jnp.int32, (BM, SCH), 1)
    liota = lax.broadcasted_iota(jnp.int32, (DL, BM, SCH), 0)
    kiota = lax.broadcasted_iota(jnp.int32, (BM, K), 1)

    def lev(l, carry):
        dw, v, i = carry
        m = jnp.min(dw, axis=0)
        sel = jnp.where(dw == m[None], iota_g, NCH)
        a = jnp.min(sel, axis=0)
        dw = jnp.where(iota_g == a[None], jnp.inf, dw)
        gi = a * SCH + lane_iota
        v = jnp.where(liota == l, m[None], v)
        i = jnp.where(liota == l, gi[None], i)
        return dw, v, i

    _, v, i = lax.fori_loop(0, DL, lev, (
        d0,
        jnp.full((DL, BM, SCH), jnp.inf, jnp.float32),
        jnp.zeros((DL, BM, SCH), jnp.int32)))

    def step(k, carry):
        v, acc, lastm, lasti = carry
        m = jnp.min(jnp.min(v, axis=0), axis=1)
        m3 = m[None, :, None]
        cand = jnp.where(v == m3, i, P)
        ii = jnp.min(jnp.min(cand, axis=0), axis=1)
        ii3 = ii[None, :, None]
        acc = jnp.where(kiota == k, ii[:, None], acc)
        v = jnp.where((v == m3) & (i == ii3), jnp.inf, v)
        return v, acc, m, ii

    _, acc, lastm, lasti = lax.fori_loop(0, K, step, (
        v, jnp.zeros((BM, K), jnp.int32),
        jnp.zeros((BM,), jnp.float32), jnp.zeros((BM,), jnp.int32)))

    lm3 = lastm[None, :, None]
    li3 = lasti[None, :, None]
    lex = (d0 < lm3) | ((d0 == lm3) & (gi4 < li3))
    cnt = jnp.sum(jnp.sum(lex.astype(jnp.int32), axis=0), axis=1)
    bad = jnp.any(cnt != K - 1)

    def _naive():
        def nstep(k, carry):
            dd, acc2 = carry
            m = jnp.min(jnp.min(dd, axis=0), axis=1)[None, :, None]
            ci = jnp.min(jnp.min(jnp.where(dd == m, gi4, P), axis=0), axis=1)
            acc2 = jnp.where(kiota == k, ci[:, None], acc2)
            dd = jnp.where(gi4 == ci[None, :, None], jnp.inf, dd)
            return dd, acc2

        return lax.fori_loop(0, K, nstep,
                             (d0, jnp.zeros((BM, K), jnp.int32)))[1]

    acc = lax.cond(bad, _naive, lambda: acc)
    b = pl.program_id(0)
    idx_ref[0] = acc + b * P

    w1t = w1t_ref[...]
    cproj = (cen[:, 0:1] * w1t[0:1, :]
             + cen[:, 1:2] * w1t[1:2, :]
             + cen[:, 2:3] * w1t[2:3, :]) - b1_ref[...]
    cproj_ref[0] = cproj

    xb = xyz_ref[0]
    fb = featsT_ref[0]
    pt = (xb[:, 0:1] * w1t[0:1, :]
          + xb[:, 1:2] * w1t[1:2, :]
          + xb[:, 2:3] * w1t[2:3, :])
    pt = pt + jnp.dot(fb, w1t[3:, :], preferred_element_type=jnp.float32)
    ptab_ref[0] = pt


def _knn_project(xyz_t, xyz, featsT, centers, w1t, b1r):
    return pl.pallas_call(
        _k1_body,
        grid=(B, M // BM),
        in_specs=[
            pl.BlockSpec((1, 3, NCH, SCH), lambda b, i: (b, 0, 0, 0)),
            pl.BlockSpec((1, PB, 3), lambda b, i: (b, i, 0)),
            pl.BlockSpec((1, PB, C_IN), lambda b, i: (b, i, 0)),
            pl.BlockSpec((1, BM, 3), lambda b, i: (b, i, 0)),
            pl.BlockSpec((3 + C_IN, C1), lambda b, i: (0, 0)),
            pl.BlockSpec((1, C1), lambda b, i: (0, 0)),
        ],
        out_specs=[
            pl.BlockSpec((1, BM, K), lambda b, i: (b, i, 0)),
            pl.BlockSpec((1, BM, C1), lambda b, i: (b, i, 0)),
            pl.BlockSpec((1, PB, C1), lambda b, i: (b, i, 0)),
        ],
        out_shape=[
            jax.ShapeDtypeStruct((B, M, K), jnp.int32),
            jax.ShapeDtypeStruct((B, M, C1), jnp.float32),
            jax.ShapeDtypeStruct((B, P, C1), jnp.float32),
        ],
    )(xyz_t, xyz, featsT, centers, w1t, b1r)


@functools.partial(
    pl.kernel,
    mesh=plsc.VectorSubcoreMesh(core_axis_name="c", subcore_axis_name="s"),
    compiler_params=pltpu.CompilerParams(use_tc_tiling_on_sc=False),
    out_type=jax.ShapeDtypeStruct((ROWS, C1), jnp.float32),
    scratch_types=[
        pltpu.VMEM((CH,), jnp.int32),
        pltpu.VMEM((CH, C1), jnp.float32),
        pltpu.SemaphoreType.DMA,
    ],
)
def _sc_gather(table_hbm, idx_hbm, out_hbm, idx_v, rows_v, sem):
    wid = lax.axis_index("s") * 2 + lax.axis_index("c")
    per_w = ROWS // NW
    base = wid * per_w

    def body(c, carry):
        off = base + c * CH
        pltpu.sync_copy(idx_hbm.at[pl.ds(off, CH)], idx_v)
        pltpu.async_copy(table_hbm.at[idx_v], rows_v, sem).wait()
        pltpu.sync_copy(rows_v, out_hbm.at[pl.ds(off, CH)])
        return carry

    lax.fori_loop(0, per_w // CH, body, 0)


def _k3_body(g_ref, cp_ref, sums_ref):
    h1 = g_ref[...] - cp_ref[...][:, None, :]
    s1 = jnp.sum(jnp.sum(h1, axis=0), axis=0)
    s2 = jnp.sum(jnp.sum(h1 * h1, axis=0), axis=0)

    @pl.when(pl.program_id(0) == 0)
    def _():
        sums_ref[...] = jnp.zeros_like(sums_ref)

    sums_ref[0:1, :] += s1[None, :]
    sums_ref[1:2, :] += s2[None, :]


def _k4_body(g_ref, cp_ref, a1_ref, s1_ref, w2t_ref, b2_ref, sums_ref):
    h1 = g_ref[...] - cp_ref[...][:, None, :]
    x1 = jnp.maximum(h1 * a1_ref[0][None, None, :]
                     + s1_ref[0][None, None, :], 0.0)
    x1f = x1.reshape(RB * K, C1)
    h2 = jnp.dot(x1f, w2t_ref[...], preferred_element_type=jnp.float32)
    h2 = h2 + b2_ref[...]
    s1 = jnp.sum(h2, axis=0)
    s2 = jnp.sum(h2 * h2, axis=0)

    @pl.when(pl.program_id(0) == 0)
    def _():
        sums_ref[...] = jnp.zeros_like(sums_ref)

    sums_ref[0:1, :] += s1[None, :]
    sums_ref[1:2, :] += s2[None, :]


def _k5_body(g_ref, cp_ref, a1_ref, s1_ref, w2t_ref, b2_ref, a2_ref, s2_ref,
             out_ref):
    h1 = g_ref[...] - cp_ref[...][:, None, :]
    x1 = jnp.maximum(h1 * a1_ref[0][None, None, :]
                     + s1_ref[0][None, None, :], 0.0)
    x1f = x1.reshape(RB * K, C1)
    h2 = jnp.dot(x1f, w2t_ref[...], preferred_element_type=jnp.float32)
    h2 = h2 + b2_ref[...]
    x2 = jnp.maximum(h2 * a2_ref[...] + s2_ref[...], 0.0)
    x3 = x2.reshape(RB, K, C2)
    mx = x3[:, 0, :]
    for k in range(1, K):
        mx = jnp.maximum(mx, x3[:, k, :])
    out_ref[...] = mx


def kernel(xyz, feats, W1, b1, g1, be1, W2, b2, g2, be2):
    idxc = jnp.linspace(0.0, P - 1, M).astype(jnp.int32)
    centers = jnp.take(xyz, idxc, axis=1)

    xyz_t = xyz.transpose(0, 2, 1).reshape(B, 3, NCH, SCH)
    featsT = feats.transpose(0, 2, 1)
    w1t = W1.T
    b1r = b1.reshape(1, C1)

    idx, cproj, ptable = _knn_project(xyz_t, xyz, featsT, centers, w1t, b1r)

    g = _sc_gather(ptable.reshape(B * P, C1), idx.reshape(ROWS))
    g3 = g.reshape(B * M, K, C1)
    cpf = cproj.reshape(B * M, C1)

    nblk = (B * M) // RB
    sums1 = pl.pallas_call(
        _k3_body,
        grid=(nblk,),
        in_specs=[
            pl.BlockSpec((RB, K, C1), lambda i: (i, 0, 0)),
            pl.BlockSpec((RB, C1), lambda i: (i, 0)),
        ],
        out_specs=pl.BlockSpec((8, C1), lambda i: (0, 0)),
        out_shape=jax.ShapeDtypeStruct((8, C1), jnp.float32),
    )(g3, cpf)

    n1 = float(ROWS)
    mean1 = sums1[0] / n1
    var1 = sums1[1] / n1 - mean1 * mean1
    sc1 = g1 / jnp.sqrt(var1 + EPS)
    sh1 = be1 - mean1 * sc1
    w2t = W2.T
    b2r = b2.reshape(1, C2)

    sums2 = pl.pallas_call(
        _k4_body,
        grid=(nblk,),
        in_specs=[
            pl.BlockSpec((RB, K, C1), lambda i: (i, 0, 0)),
            pl.BlockSpec((RB, C1), lambda i: (i, 0)),
            pl.BlockSpec((1, C1), lambda i: (0, 0)),
            pl.BlockSpec((1, C1), lambda i: (0, 0)),
            pl.BlockSpec((C1, C2), lambda i: (0, 0)),
            pl.BlockSpec((1, C2), lambda i: (0, 0)),
        ],
        out_specs=pl.BlockSpec((8, C2), lambda i: (0, 0)),
        out_shape=jax.ShapeDtypeStruct((8, C2), jnp.float32),
    )(g3, cpf, sc1.reshape(1, C1), sh1.reshape(1, C1), w2t, b2r)

    mean2 = sums2[0] / n1
    var2 = sums2[1] / n1 - mean2 * mean2
    sc2 = g2 / jnp.sqrt(var2 + EPS)
    sh2 = be2 - mean2 * sc2

    out2 = pl.pallas_call(
        _k5_body,
        grid=(nblk,),
        in_specs=[
            pl.BlockSpec((RB, K, C1), lambda i: (i, 0, 0)),
            pl.BlockSpec((RB, C1), lambda i: (i, 0)),
            pl.BlockSpec((1, C1), lambda i: (0, 0)),
            pl.BlockSpec((1, C1), lambda i: (0, 0)),
            pl.BlockSpec((C1, C2), lambda i: (0, 0)),
            pl.BlockSpec((1, C2), lambda i: (0, 0)),
            pl.BlockSpec((1, C2), lambda i: (0, 0)),
            pl.BlockSpec((1, C2), lambda i: (0, 0)),
        ],
        out_specs=pl.BlockSpec((RB, C2), lambda i: (i, 0)),
        out_shape=jax.ShapeDtypeStruct((B * M, C2), jnp.float32),
    )(g3, cpf, sc1.reshape(1, C1), sh1.reshape(1, C1), w2t, b2r,
      sc2.reshape(1, C2), sh2.reshape(1, C2))

    out = out2.reshape(B, M, C2).transpose(0, 2, 1)
    return centers, out


# speedup vs baseline: 2.5882x; 1.1941x over previous
import functools

import jax
import jax.numpy as jnp
from jax import lax
from jax.experimental import pallas as pl
from jax.experimental.pallas import tpu as pltpu
from jax.experimental.pallas import tpu_sc as plsc

B, P, C_IN = 4, 8192, 16
M = P // 4
K = 32
C1, C2 = 32, 64
BM = 128
PB = P // (M // BM)
RB = 256
NW = 32
ROWS = B * M * K
CH = 128
EPS = 1e-5


NCH = 64
SCH = 128
DL = 6


def _k1_body(xyzt_ref, xyz_ref, featsT_ref, cen_ref, w1t_ref, b1_ref,
             idx_ref, cproj_ref, ptab_ref):
    xt4 = xyzt_ref[0]
    cen = cen_ref[0]
    pn = jnp.sum(xt4 * xt4, axis=0)
    cn = jnp.sum(cen * cen, axis=1)
    cb = cen.astype(jnp.bfloat16).astype(jnp.float32)
    xb = xt4.astype(jnp.bfloat16).astype(jnp.float32)
    dot = (cb[:, 0][None, :, None] * xb[0][:, None, :]
           + cb[:, 1][None, :, None] * xb[1][:, None, :]
           + cb[:, 2][None, :, None] * xb[2][:, None, :])
    d0 = cn[None, :, None] + pn[:, None, :] - 2.0 * dot

    iota_g = lax.broadcasted_iota(jnp.int32, (NCH, BM, SCH), 0)
    iota_l = lax.broadcasted_iota(jnp.int32, (NCH, BM, SCH), 2)
    gi4 = iota_g * SCH + iota_l
    lane_iota = lax.broadcasted_iota(jnp.int32, (BM, SCH), 1)
    liota = lax.broadcasted_iota(jnp.int32, (DL, BM, SCH), 0)
    kiota = lax.broadcasted_iota(jnp.int32, (BM, K), 1)

    def lev(l, carry):
        dw, v, i = carry
        m = jnp.min(dw, axis=0)
        sel = jnp.where(dw == m[None], iota_g, NCH)
        a = jnp.min(sel, axis=0)
        dw = jnp.where(iota_g == a[None], jnp.inf, dw)
        gi = a * SCH + lane_iota
        v = jnp.where(liota == l, m[None], v)
        i = jnp.where(liota == l, gi[None], i)
        return dw, v, i

    _, v, i = lax.fori_loop(0, DL, lev, (
        d0,
        jnp.full((DL, BM, SCH), jnp.inf, jnp.float32),
        jnp.zeros((DL, BM, SCH), jnp.int32)))

    def step(k, carry):
        v, acc, lastm, lasti = carry
        m = jnp.min(jnp.min(v, axis=0), axis=1)
        m3 = m[None, :, None]
        cand = jnp.where(v == m3, i, P)
        ii = jnp.min(jnp.min(cand, axis=0), axis=1)
        ii3 = ii[None, :, None]
        acc = jnp.where(kiota == k, ii[:, None], acc)
        v = jnp.where((v == m3) & (i == ii3), jnp.inf, v)
        return v, acc, m, ii

    _, acc, lastm, lasti = lax.fori_loop(0, K, step, (
        v, jnp.zeros((BM, K), jnp.int32),
        jnp.zeros((BM,), jnp.float32), jnp.zeros((BM,), jnp.int32)))

    lm3 = lastm[None, :, None]
    li3 = lasti[None, :, None]
    lex = (d0 < lm3) | ((d0 == lm3) & (gi4 < li3))
    cnt = jnp.sum(jnp.sum(lex.astype(jnp.int32), axis=0), axis=1)
    bad = jnp.any(cnt != K - 1)

    def _naive():
        def nstep(k, carry):
            dd, acc2 = carry
            m = jnp.min(jnp.min(dd, axis=0), axis=1)[None, :, None]
            ci = jnp.min(jnp.min(jnp.where(dd == m, gi4, P), axis=0), axis=1)
            acc2 = jnp.where(kiota == k, ci[:, None], acc2)
            dd = jnp.where(gi4 == ci[None, :, None], jnp.inf, dd)
            return dd, acc2

        return lax.fori_loop(0, K, nstep,
                             (d0, jnp.zeros((BM, K), jnp.int32)))[1]

    acc = lax.cond(bad, _naive, lambda: acc)
    b = pl.program_id(0)
    idx_ref[0] = acc + b * P

    w1t = w1t_ref[...]
    cproj = (cen[:, 0:1] * w1t[0:1, :]
             + cen[:, 1:2] * w1t[1:2, :]
             + cen[:, 2:3] * w1t[2:3, :]) - b1_ref[...]
    cproj_ref[0] = cproj

    xb = xyz_ref[0]
    fb = featsT_ref[0]
    pt = (xb[:, 0:1] * w1t[0:1, :]
          + xb[:, 1:2] * w1t[1:2, :]
          + xb[:, 2:3] * w1t[2:3, :])
    pt = pt + jnp.dot(fb, w1t[3:, :], preferred_element_type=jnp.float32)
    ptab_ref[0] = pt


def _knn_project(xyz_t, xyz, featsT, centers, w1t, b1r):
    return pl.pallas_call(
        _k1_body,
        grid=(B, M // BM),
        in_specs=[
            pl.BlockSpec((1, 3, NCH, SCH), lambda b, i: (b, 0, 0, 0)),
            pl.BlockSpec((1, PB, 3), lambda b, i: (b, i, 0)),
            pl.BlockSpec((1, PB, C_IN), lambda b, i: (b, i, 0)),
            pl.BlockSpec((1, BM, 3), lambda b, i: (b, i, 0)),
            pl.BlockSpec((3 + C_IN, C1), lambda b, i: (0, 0)),
            pl.BlockSpec((1, C1), lambda b, i: (0, 0)),
        ],
        out_specs=[
            pl.BlockSpec((1, BM, K), lambda b, i: (b, i, 0)),
            pl.BlockSpec((1, BM, C1), lambda b, i: (b, i, 0)),
            pl.BlockSpec((1, PB, C1), lambda b, i: (b, i, 0)),
        ],
        out_shape=[
            jax.ShapeDtypeStruct((B, M, K), jnp.int32),
            jax.ShapeDtypeStruct((B, M, C1), jnp.float32),
            jax.ShapeDtypeStruct((B, P, C1), jnp.float32),
        ],
    )(xyz_t, xyz, featsT, centers, w1t, b1r)


@functools.partial(
    pl.kernel,
    mesh=plsc.VectorSubcoreMesh(core_axis_name="c", subcore_axis_name="s"),
    compiler_params=pltpu.CompilerParams(use_tc_tiling_on_sc=False),
    out_type=jax.ShapeDtypeStruct((ROWS, C1), jnp.float32),
    scratch_types=[
        pltpu.VMEM((CH,), jnp.int32),
        pltpu.VMEM((CH, C1), jnp.float32),
        pltpu.SemaphoreType.DMA,
    ],
)
def _sc_gather(table_hbm, idx_hbm, out_hbm, idx_v, rows_v, sem):
    wid = lax.axis_index("s") * 2 + lax.axis_index("c")
    per_w = ROWS // NW
    base = wid * per_w

    def body(c, carry):
        off = base + c * CH
        pltpu.sync_copy(idx_hbm.at[pl.ds(off, CH)], idx_v)
        pltpu.async_copy(table_hbm.at[idx_v], rows_v, sem).wait()
        pltpu.sync_copy(rows_v, out_hbm.at[pl.ds(off, CH)])
        return carry

    lax.fori_loop(0, per_w // CH, body, 0)


def _k3_body(g_ref, cp_ref, sums_ref):
    h1 = g_ref[...] - cp_ref[...][:, None, :]
    s1 = jnp.sum(jnp.sum(h1, axis=0), axis=0)
    s2 = jnp.sum(jnp.sum(h1 * h1, axis=0), axis=0)

    @pl.when(pl.program_id(0) == 0)
    def _():
        sums_ref[...] = jnp.zeros_like(sums_ref)

    sums_ref[0:1, :] += s1[None, :]
    sums_ref[1:2, :] += s2[None, :]


def _k4_body(g_ref, cp_ref, a1_ref, s1_ref, w2t_ref, b2_ref, sums_ref):
    h1 = g_ref[...] - cp_ref[...][:, None, :]
    x1 = jnp.maximum(h1 * a1_ref[0][None, None, :]
                     + s1_ref[0][None, None, :], 0.0)
    x1f = x1.reshape(RB * K, C1)
    h2 = jnp.dot(x1f, w2t_ref[...], preferred_element_type=jnp.float32)
    h2 = h2 + b2_ref[...]
    s1 = jnp.sum(h2, axis=0)
    s2 = jnp.sum(h2 * h2, axis=0)

    @pl.when(pl.program_id(0) == 0)
    def _():
        sums_ref[...] = jnp.zeros_like(sums_ref)

    sums_ref[0:1, :] += s1[None, :]
    sums_ref[1:2, :] += s2[None, :]


def _k5_body(g_ref, cp_ref, a1_ref, s1_ref, w2t_ref, b2_ref, a2_ref, s2_ref,
             out_ref):
    h1 = g_ref[...] - cp_ref[...][:, None, :]
    x1 = jnp.maximum(h1 * a1_ref[0][None, None, :]
                     + s1_ref[0][None, None, :], 0.0)
    x1f = x1.reshape(RB * K, C1)
    h2 = jnp.dot(x1f, w2t_ref[...], preferred_element_type=jnp.float32)
    h2 = h2 + b2_ref[...]
    x2 = jnp.maximum(h2 * a2_ref[...] + s2_ref[...], 0.0)
    x3 = x2.reshape(RB, K, C2)
    mx = x3[:, 0, :]
    for k in range(1, K):
        mx = jnp.maximum(mx, x3[:, k, :])
    out_ref[...] = mx


def kernel(xyz, feats, W1, b1, g1, be1, W2, b2, g2, be2):
    idxc = jnp.linspace(0.0, P - 1, M).astype(jnp.int32)
    centers = jnp.take(xyz, idxc, axis=1)

    xyz_t = xyz.transpose(0, 2, 1).reshape(B, 3, NCH, SCH)
    featsT = feats.transpose(0, 2, 1)
    w1t = W1.T
    b1r = b1.reshape(1, C1)

    idx, cproj, ptable = _knn_project(xyz_t, xyz, featsT, centers, w1t, b1r)

    g = _sc_gather(ptable.reshape(B * P, C1), idx.reshape(ROWS))
    g3 = g.reshape(B * M, K, C1)
    cpf = cproj.reshape(B * M, C1)

    nblk = (B * M) // RB
    sums1 = pl.pallas_call(
        _k3_body,
        grid=(nblk,),
        in_specs=[
            pl.BlockSpec((RB, K, C1), lambda i: (i, 0, 0)),
            pl.BlockSpec((RB, C1), lambda i: (i, 0)),
        ],
        out_specs=pl.BlockSpec((8, C1), lambda i: (0, 0)),
        out_shape=jax.ShapeDtypeStruct((8, C1), jnp.float32),
    )(g3, cpf)

    n1 = float(ROWS)
    mean1 = sums1[0] / n1
    var1 = sums1[1] / n1 - mean1 * mean1
    sc1 = g1 / jnp.sqrt(var1 + EPS)
    sh1 = be1 - mean1 * sc1
    w2t = W2.T
    b2r = b2.reshape(1, C2)

    sums2 = pl.pallas_call(
        _k4_body,
        grid=(nblk,),
        in_specs=[
            pl.BlockSpec((RB, K, C1), lambda i: (i, 0, 0)),
            pl.BlockSpec((RB, C1), lambda i: (i, 0)),
            pl.BlockSpec((1, C1), lambda i: (0, 0)),
            pl.BlockSpec((1, C1), lambda i: (0, 0)),
            pl.BlockSpec((C1, C2), lambda i: (0, 0)),
            pl.BlockSpec((1, C2), lambda i: (0, 0)),
        ],
        out_specs=pl.BlockSpec((8, C2), lambda i: (0, 0)),
        out_shape=jax.ShapeDtypeStruct((8, C2), jnp.float32),
    )(g3, cpf, sc1.reshape(1, C1), sh1.reshape(1, C1), w2t, b2r)

    mean2 = sums2[0] / n1
    var2 = sums2[1] / n1 - mean2 * mean2
    sc2 = g2 / jnp.sqrt(var2 + EPS)
    sh2 = be2 - mean2 * sc2

    out2 = pl.pallas_call(
        _k5_body,
        grid=(nblk,),
        in_specs=[
            pl.BlockSpec((RB, K, C1), lambda i: (i, 0, 0)),
            pl.BlockSpec((RB, C1), lambda i: (i, 0)),
            pl.BlockSpec((1, C1), lambda i: (0, 0)),
            pl.BlockSpec((1, C1), lambda i: (0, 0)),
            pl.BlockSpec((C1, C2), lambda i: (0, 0)),
            pl.BlockSpec((1, C2), lambda i: (0, 0)),
            pl.BlockSpec((1, C2), lambda i: (0, 0)),
            pl.BlockSpec((1, C2), lambda i: (0, 0)),
        ],
        out_specs=pl.BlockSpec((RB, C2), lambda i: (i, 0)),
        out_shape=jax.ShapeDtypeStruct((B * M, C2), jnp.float32),
    )(g3, cpf, sc1.reshape(1, C1), sh1.reshape(1, C1), w2t, b2r,
      sc2.reshape(1, C2), sh2.reshape(1, C2))

    out = out2.reshape(B, M, C2).transpose(0, 2, 1)
    return centers, out


# speedup vs baseline: 2.8675x; 1.1079x over previous
import functools

import jax
import jax.numpy as jnp
from jax import lax
from jax.experimental import pallas as pl
from jax.experimental.pallas import tpu as pltpu
from jax.experimental.pallas import tpu_sc as plsc

B, P, C_IN = 4, 8192, 16
M = P // 4
K = 32
C1, C2 = 32, 64
BM = 256
PB = P // (M // BM)
RB = 256
NW = 32
ROWS = B * M * K
CH = 128
EPS = 1e-5


NCH = 64
SCH = 128
DL = 6


def _k1_body(xyzt_ref, xyz_ref, featsT_ref, cen_ref, w1t_ref, b1_ref,
             idx_ref, cproj_ref, ptab_ref):
    xt4 = xyzt_ref[0]
    cen = cen_ref[0]
    pn = jnp.sum(xt4 * xt4, axis=0)
    cn = jnp.sum(cen * cen, axis=1)
    cb = cen.astype(jnp.bfloat16).astype(jnp.float32)
    xb = xt4.astype(jnp.bfloat16).astype(jnp.float32)
    dot = (cb[:, 0][None, :, None] * xb[0][:, None, :]
           + cb[:, 1][None, :, None] * xb[1][:, None, :]
           + cb[:, 2][None, :, None] * xb[2][:, None, :])
    d0 = cn[None, :, None] + pn[:, None, :] - 2.0 * dot

    iota_g = lax.broadcasted_iota(jnp.int32, (NCH, BM, SCH), 0)
    iota_l = lax.broadcasted_iota(jnp.int32, (NCH, BM, SCH), 2)
    gi4 = iota_g * SCH + iota_l
    lane_iota = lax.broadcasted_iota(jnp.int32, (BM, SCH), 1)
    liota = lax.broadcasted_iota(jnp.int32, (DL, BM, SCH), 0)
    kiota = lax.broadcasted_iota(jnp.int32, (BM, K), 1)

    def lev(l, carry):
        dw, v, i = carry
        m = jnp.min(dw, axis=0)
        sel = jnp.where(dw == m[None], iota_g, NCH)
        a = jnp.min(sel, axis=0)
        dw = jnp.where(iota_g == a[None], jnp.inf, dw)
        gi = a * SCH + lane_iota
        v = jnp.where(liota == l, m[None], v)
        i = jnp.where(liota == l, gi[None], i)
        return dw, v, i

    _, v, i = lax.fori_loop(0, DL, lev, (
        d0,
        jnp.full((DL, BM, SCH), jnp.inf, jnp.float32),
        jnp.zeros((DL, BM, SCH), jnp.int32)))

    def step(k, carry):
        v, acc, lastm, lasti = carry
        m = jnp.min(jnp.min(v, axis=0), axis=1)
        m3 = m[None, :, None]
        cand = jnp.where(v == m3, i, P)
        ii = jnp.min(jnp.min(cand, axis=0), axis=1)
        ii3 = ii[None, :, None]
        acc = jnp.where(kiota == k, ii[:, None], acc)
        v = jnp.where((v == m3) & (i == ii3), jnp.inf, v)
        return v, acc, m, ii

    _, acc, lastm, lasti = lax.fori_loop(0, K, step, (
        v, jnp.zeros((BM, K), jnp.int32),
        jnp.zeros((BM,), jnp.float32), jnp.zeros((BM,), jnp.int32)))

    lm3 = lastm[None, :, None]
    li3 = lasti[None, :, None]
    lex = (d0 < lm3) | ((d0 == lm3) & (gi4 < li3))
    cnt = jnp.sum(jnp.sum(lex.astype(jnp.int32), axis=0), axis=1)
    bad = jnp.any(cnt != K - 1)

    def _naive():
        def nstep(k, carry):
            dd, acc2 = carry
            m = jnp.min(jnp.min(dd, axis=0), axis=1)[None, :, None]
            ci = jnp.min(jnp.min(jnp.where(dd == m, gi4, P), axis=0), axis=1)
            acc2 = jnp.where(kiota == k, ci[:, None], acc2)
            dd = jnp.where(gi4 == ci[None, :, None], jnp.inf, dd)
            return dd, acc2

        return lax.fori_loop(0, K, nstep,
                             (d0, jnp.zeros((BM, K), jnp.int32)))[1]

    acc = lax.cond(bad, _naive, lambda: acc)
    b = pl.program_id(0)
    idx_ref[0] = acc + b * P

    w1t = w1t_ref[...]
    cproj = (cen[:, 0:1] * w1t[0:1, :]
             + cen[:, 1:2] * w1t[1:2, :]
             + cen[:, 2:3] * w1t[2:3, :]) - b1_ref[...]
    cproj_ref[0] = cproj

    xb = xyz_ref[0]
    fb = featsT_ref[0]
    pt = (xb[:, 0:1] * w1t[0:1, :]
          + xb[:, 1:2] * w1t[1:2, :]
          + xb[:, 2:3] * w1t[2:3, :])
    pt = pt + jnp.dot(fb, w1t[3:, :], preferred_element_type=jnp.float32)
    ptab_ref[0] = pt


def _knn_project(xyz_t, xyz, featsT, centers, w1t, b1r):
    return pl.pallas_call(
        _k1_body,
        grid=(B, M // BM),
        in_specs=[
            pl.BlockSpec((1, 3, NCH, SCH), lambda b, i: (b, 0, 0, 0)),
            pl.BlockSpec((1, PB, 3), lambda b, i: (b, i, 0)),
            pl.BlockSpec((1, PB, C_IN), lambda b, i: (b, i, 0)),
            pl.BlockSpec((1, BM, 3), lambda b, i: (b, i, 0)),
            pl.BlockSpec((3 + C_IN, C1), lambda b, i: (0, 0)),
            pl.BlockSpec((1, C1), lambda b, i: (0, 0)),
        ],
        out_specs=[
            pl.BlockSpec((1, BM, K), lambda b, i: (b, i, 0)),
            pl.BlockSpec((1, BM, C1), lambda b, i: (b, i, 0)),
            pl.BlockSpec((1, PB, C1), lambda b, i: (b, i, 0)),
        ],
        out_shape=[
            jax.ShapeDtypeStruct((B, M, K), jnp.int32),
            jax.ShapeDtypeStruct((B, M, C1), jnp.float32),
            jax.ShapeDtypeStruct((B, P, C1), jnp.float32),
        ],
    )(xyz_t, xyz, featsT, centers, w1t, b1r)


@functools.partial(
    pl.kernel,
    mesh=plsc.VectorSubcoreMesh(core_axis_name="c", subcore_axis_name="s"),
    compiler_params=pltpu.CompilerParams(use_tc_tiling_on_sc=False),
    out_type=jax.ShapeDtypeStruct((ROWS, C1), jnp.float32),
    scratch_types=[
        pltpu.VMEM((CH,), jnp.int32),
        pltpu.VMEM((CH, C1), jnp.float32),
        pltpu.SemaphoreType.DMA,
    ],
)
def _sc_gather(table_hbm, idx_hbm, out_hbm, idx_v, rows_v, sem):
    wid = lax.axis_index("s") * 2 + lax.axis_index("c")
    per_w = ROWS // NW
    base = wid * per_w

    def body(c, carry):
        off = base + c * CH
        pltpu.sync_copy(idx_hbm.at[pl.ds(off, CH)], idx_v)
        pltpu.async_copy(table_hbm.at[idx_v], rows_v, sem).wait()
        pltpu.sync_copy(rows_v, out_hbm.at[pl.ds(off, CH)])
        return carry

    lax.fori_loop(0, per_w // CH, body, 0)


def _k3_body(g_ref, cp_ref, sums_ref):
    h1 = g_ref[...] - cp_ref[...][:, None, :]
    s1 = jnp.sum(jnp.sum(h1, axis=0), axis=0)
    s2 = jnp.sum(jnp.sum(h1 * h1, axis=0), axis=0)

    @pl.when(pl.program_id(0) == 0)
    def _():
        sums_ref[...] = jnp.zeros_like(sums_ref)

    sums_ref[0:1, :] += s1[None, :]
    sums_ref[1:2, :] += s2[None, :]


def _k4_body(g_ref, cp_ref, a1_ref, s1_ref, w2t_ref, b2_ref, sums_ref):
    h1 = g_ref[...] - cp_ref[...][:, None, :]
    x1 = jnp.maximum(h1 * a1_ref[0][None, None, :]
                     + s1_ref[0][None, None, :], 0.0)
    x1f = x1.reshape(RB * K, C1)
    h2 = jnp.dot(x1f, w2t_ref[...], preferred_element_type=jnp.float32)
    h2 = h2 + b2_ref[...]
    s1 = jnp.sum(h2, axis=0)
    s2 = jnp.sum(h2 * h2, axis=0)

    @pl.when(pl.program_id(0) == 0)
    def _():
        sums_ref[...] = jnp.zeros_like(sums_ref)

    sums_ref[0:1, :] += s1[None, :]
    sums_ref[1:2, :] += s2[None, :]


def _k5_body(g_ref, cp_ref, a1_ref, s1_ref, w2t_ref, b2_ref, a2_ref, s2_ref,
             out_ref):
    h1 = g_ref[...] - cp_ref[...][:, None, :]
    x1 = jnp.maximum(h1 * a1_ref[0][None, None, :]
                     + s1_ref[0][None, None, :], 0.0)
    x1f = x1.reshape(RB * K, C1)
    h2 = jnp.dot(x1f, w2t_ref[...], preferred_element_type=jnp.float32)
    h2 = h2 + b2_ref[...]
    x2 = jnp.maximum(h2 * a2_ref[...] + s2_ref[...], 0.0)
    x3 = x2.reshape(RB, K, C2)
    mx = x3[:, 0, :]
    for k in range(1, K):
        mx = jnp.maximum(mx, x3[:, k, :])
    out_ref[...] = mx


def kernel(xyz, feats, W1, b1, g1, be1, W2, b2, g2, be2):
    idxc = jnp.linspace(0.0, P - 1, M).astype(jnp.int32)
    centers = jnp.take(xyz, idxc, axis=1)

    xyz_t = xyz.transpose(0, 2, 1).reshape(B, 3, NCH, SCH)
    featsT = feats.transpose(0, 2, 1)
    w1t = W1.T
    b1r = b1.reshape(1, C1)

    idx, cproj, ptable = _knn_project(xyz_t, xyz, featsT, centers, w1t, b1r)

    g = _sc_gather(ptable.reshape(B * P, C1), idx.reshape(ROWS))
    g3 = g.reshape(B * M, K, C1)
    cpf = cproj.reshape(B * M, C1)

    nblk = (B * M) // RB
    sums1 = pl.pallas_call(
        _k3_body,
        grid=(nblk,),
        in_specs=[
            pl.BlockSpec((RB, K, C1), lambda i: (i, 0, 0)),
            pl.BlockSpec((RB, C1), lambda i: (i, 0)),
        ],
        out_specs=pl.BlockSpec((8, C1), lambda i: (0, 0)),
        out_shape=jax.ShapeDtypeStruct((8, C1), jnp.float32),
    )(g3, cpf)

    n1 = float(ROWS)
    mean1 = sums1[0] / n1
    var1 = sums1[1] / n1 - mean1 * mean1
    sc1 = g1 / jnp.sqrt(var1 + EPS)
    sh1 = be1 - mean1 * sc1
    w2t = W2.T
    b2r = b2.reshape(1, C2)

    sums2 = pl.pallas_call(
        _k4_body,
        grid=(nblk,),
        in_specs=[
            pl.BlockSpec((RB, K, C1), lambda i: (i, 0, 0)),
            pl.BlockSpec((RB, C1), lambda i: (i, 0)),
            pl.BlockSpec((1, C1), lambda i: (0, 0)),
            pl.BlockSpec((1, C1), lambda i: (0, 0)),
            pl.BlockSpec((C1, C2), lambda i: (0, 0)),
            pl.BlockSpec((1, C2), lambda i: (0, 0)),
        ],
        out_specs=pl.BlockSpec((8, C2), lambda i: (0, 0)),
        out_shape=jax.ShapeDtypeStruct((8, C2), jnp.float32),
    )(g3, cpf, sc1.reshape(1, C1), sh1.reshape(1, C1), w2t, b2r)

    mean2 = sums2[0] / n1
    var2 = sums2[1] / n1 - mean2 * mean2
    sc2 = g2 / jnp.sqrt(var2 + EPS)
    sh2 = be2 - mean2 * sc2

    out2 = pl.pallas_call(
        _k5_body,
        grid=(nblk,),
        in_specs=[
            pl.BlockSpec((RB, K, C1), lambda i: (i, 0, 0)),
            pl.BlockSpec((RB, C1), lambda i: (i, 0)),
            pl.BlockSpec((1, C1), lambda i: (0, 0)),
            pl.BlockSpec((1, C1), lambda i: (0, 0)),
            pl.BlockSpec((C1, C2), lambda i: (0, 0)),
            pl.BlockSpec((1, C2), lambda i: (0, 0)),
            pl.BlockSpec((1, C2), lambda i: (0, 0)),
            pl.BlockSpec((1, C2), lambda i: (0, 0)),
        ],
        out_specs=pl.BlockSpec((RB, C2), lambda i: (i, 0)),
        out_shape=jax.ShapeDtypeStruct((B * M, C2), jnp.float32),
    )(g3, cpf, sc1.reshape(1, C1), sh1.reshape(1, C1), w2t, b2r,
      sc2.reshape(1, C2), sh2.reshape(1, C2))

    out = out2.reshape(B, M, C2).transpose(0, 2, 1)
    return centers, out


# speedup vs baseline: 3.1662x; 1.1042x over previous
import functools

import jax
import jax.numpy as jnp
from jax import lax
from jax.experimental import pallas as pl
from jax.experimental.pallas import tpu as pltpu
from jax.experimental.pallas import tpu_sc as plsc

B, P, C_IN = 4, 8192, 16
M = P // 4
K = 32
C1, C2 = 32, 64
BM = 256
PB = P // (M // BM)
RB = 256
NW = 32
ROWS = B * M * K
CH = 128
EPS = 1e-5


NCH = 64
SCH = 128
DL = 5


def _k1_body(xyzt_ref, xyz_ref, featsT_ref, cen_ref, w1t_ref, b1_ref,
             idx_ref, cproj_ref, ptab_ref):
    xt4 = xyzt_ref[0]
    cen = cen_ref[0]
    pn = jnp.sum(xt4 * xt4, axis=0)
    cn = jnp.sum(cen * cen, axis=1)
    cb = cen.astype(jnp.bfloat16).astype(jnp.float32)
    xb = xt4.astype(jnp.bfloat16).astype(jnp.float32)
    dot = (cb[:, 0][None, :, None] * xb[0][:, None, :]
           + cb[:, 1][None, :, None] * xb[1][:, None, :]
           + cb[:, 2][None, :, None] * xb[2][:, None, :])
    d0 = cn[None, :, None] + pn[:, None, :] - 2.0 * dot

    iota_g = lax.broadcasted_iota(jnp.int32, (NCH, BM, SCH), 0)
    iota_l = lax.broadcasted_iota(jnp.int32, (NCH, BM, SCH), 2)
    gi4 = iota_g * SCH + iota_l
    lane_iota = lax.broadcasted_iota(jnp.int32, (BM, SCH), 1)
    liota = lax.broadcasted_iota(jnp.int32, (DL, BM, SCH), 0)
    kiota = lax.broadcasted_iota(jnp.int32, (BM, K), 1)

    def lev(l, carry):
        dw, v, i = carry
        m = jnp.min(dw, axis=0)
        sel = jnp.where(dw == m[None], iota_g, NCH)
        a = jnp.min(sel, axis=0)
        dw = jnp.where(iota_g == a[None], jnp.inf, dw)
        gi = a * SCH + lane_iota
        v = jnp.where(liota == l, m[None], v)
        i = jnp.where(liota == l, gi[None], i)
        return dw, v, i

    _, v, i = lax.fori_loop(0, DL, lev, (
        d0,
        jnp.full((DL, BM, SCH), jnp.inf, jnp.float32),
        jnp.zeros((DL, BM, SCH), jnp.int32)))

    def step(k, carry):
        v, acc, lastm, lasti = carry
        m = jnp.min(jnp.min(v, axis=0), axis=1)
        m3 = m[None, :, None]
        cand = jnp.where(v == m3, i, P)
        ii = jnp.min(jnp.min(cand, axis=0), axis=1)
        ii3 = ii[None, :, None]
        acc = jnp.where(kiota == k, ii[:, None], acc)
        v = jnp.where((v == m3) & (i == ii3), jnp.inf, v)
        return v, acc, m, ii

    _, acc, lastm, lasti = lax.fori_loop(0, K, step, (
        v, jnp.zeros((BM, K), jnp.int32),
        jnp.zeros((BM,), jnp.float32), jnp.zeros((BM,), jnp.int32)))

    lm3 = lastm[None, :, None]
    li3 = lasti[None, :, None]
    lex = (d0 < lm3) | ((d0 == lm3) & (gi4 < li3))
    cnt = jnp.sum(jnp.sum(lex.astype(jnp.int32), axis=0), axis=1)
    bad = jnp.any(cnt != K - 1)

    def _naive():
        def nstep(k, carry):
            dd, acc2 = carry
            m = jnp.min(jnp.min(dd, axis=0), axis=1)[None, :, None]
            ci = jnp.min(jnp.min(jnp.where(dd == m, gi4, P), axis=0), axis=1)
            acc2 = jnp.where(kiota == k, ci[:, None], acc2)
            dd = jnp.where(gi4 == ci[None, :, None], jnp.inf, dd)
            return dd, acc2

        return lax.fori_loop(0, K, nstep,
                             (d0, jnp.zeros((BM, K), jnp.int32)))[1]

    acc = lax.cond(bad, _naive, lambda: acc)
    b = pl.program_id(0)
    idx_ref[0] = acc + b * P

    w1t = w1t_ref[...]
    cproj = (cen[:, 0:1] * w1t[0:1, :]
             + cen[:, 1:2] * w1t[1:2, :]
             + cen[:, 2:3] * w1t[2:3, :]) - b1_ref[...]
    cproj_ref[0] = cproj

    xb = xyz_ref[0]
    fb = featsT_ref[0]
    pt = (xb[:, 0:1] * w1t[0:1, :]
          + xb[:, 1:2] * w1t[1:2, :]
          + xb[:, 2:3] * w1t[2:3, :])
    pt = pt + jnp.dot(fb, w1t[3:, :], preferred_element_type=jnp.float32)
    ptab_ref[0] = pt


def _knn_project(xyz_t, xyz, featsT, centers, w1t, b1r):
    return pl.pallas_call(
        _k1_body,
        grid=(B, M // BM),
        in_specs=[
            pl.BlockSpec((1, 3, NCH, SCH), lambda b, i: (b, 0, 0, 0)),
            pl.BlockSpec((1, PB, 3), lambda b, i: (b, i, 0)),
            pl.BlockSpec((1, PB, C_IN), lambda b, i: (b, i, 0)),
            pl.BlockSpec((1, BM, 3), lambda b, i: (b, i, 0)),
            pl.BlockSpec((3 + C_IN, C1), lambda b, i: (0, 0)),
            pl.BlockSpec((1, C1), lambda b, i: (0, 0)),
        ],
        out_specs=[
            pl.BlockSpec((1, BM, K), lambda b, i: (b, i, 0)),
            pl.BlockSpec((1, BM, C1), lambda b, i: (b, i, 0)),
            pl.BlockSpec((1, PB, C1), lambda b, i: (b, i, 0)),
        ],
        out_shape=[
            jax.ShapeDtypeStruct((B, M, K), jnp.int32),
            jax.ShapeDtypeStruct((B, M, C1), jnp.float32),
            jax.ShapeDtypeStruct((B, P, C1), jnp.float32),
        ],
    )(xyz_t, xyz, featsT, centers, w1t, b1r)


@functools.partial(
    pl.kernel,
    mesh=plsc.VectorSubcoreMesh(core_axis_name="c", subcore_axis_name="s"),
    compiler_params=pltpu.CompilerParams(use_tc_tiling_on_sc=False),
    out_type=jax.ShapeDtypeStruct((ROWS, C1), jnp.float32),
    scratch_types=[
        pltpu.VMEM((CH,), jnp.int32),
        pltpu.VMEM((CH, C1), jnp.float32),
        pltpu.SemaphoreType.DMA,
    ],
)
def _sc_gather(table_hbm, idx_hbm, out_hbm, idx_v, rows_v, sem):
    wid = lax.axis_index("s") * 2 + lax.axis_index("c")
    per_w = ROWS // NW
    base = wid * per_w

    def body(c, carry):
        off = base + c * CH
        pltpu.sync_copy(idx_hbm.at[pl.ds(off, CH)], idx_v)
        pltpu.async_copy(table_hbm.at[idx_v], rows_v, sem).wait()
        pltpu.sync_copy(rows_v, out_hbm.at[pl.ds(off, CH)])
        return carry

    lax.fori_loop(0, per_w // CH, body, 0)


def _k3_body(g_ref, cp_ref, sums_ref):
    h1 = g_ref[...] - cp_ref[...][:, None, :]
    s1 = jnp.sum(jnp.sum(h1, axis=0), axis=0)
    s2 = jnp.sum(jnp.sum(h1 * h1, axis=0), axis=0)

    @pl.when(pl.program_id(0) == 0)
    def _():
        sums_ref[...] = jnp.zeros_like(sums_ref)

    sums_ref[0:1, :] += s1[None, :]
    sums_ref[1:2, :] += s2[None, :]


def _k4_body(g_ref, cp_ref, a1_ref, s1_ref, w2t_ref, b2_ref, sums_ref):
    h1 = g_ref[...] - cp_ref[...][:, None, :]
    x1 = jnp.maximum(h1 * a1_ref[0][None, None, :]
                     + s1_ref[0][None, None, :], 0.0)
    x1f = x1.reshape(RB * K, C1)
    h2 = jnp.dot(x1f, w2t_ref[...], preferred_element_type=jnp.float32)
    h2 = h2 + b2_ref[...]
    s1 = jnp.sum(h2, axis=0)
    s2 = jnp.sum(h2 * h2, axis=0)

    @pl.when(pl.program_id(0) == 0)
    def _():
        sums_ref[...] = jnp.zeros_like(sums_ref)

    sums_ref[0:1, :] += s1[None, :]
    sums_ref[1:2, :] += s2[None, :]


def _k5_body(g_ref, cp_ref, a1_ref, s1_ref, w2t_ref, b2_ref, a2_ref, s2_ref,
             out_ref):
    h1 = g_ref[...] - cp_ref[...][:, None, :]
    x1 = jnp.maximum(h1 * a1_ref[0][None, None, :]
                     + s1_ref[0][None, None, :], 0.0)
    x1f = x1.reshape(RB * K, C1)
    h2 = jnp.dot(x1f, w2t_ref[...], preferred_element_type=jnp.float32)
    h2 = h2 + b2_ref[...]
    x2 = jnp.maximum(h2 * a2_ref[...] + s2_ref[...], 0.0)
    x3 = x2.reshape(RB, K, C2)
    mx = x3[:, 0, :]
    for k in range(1, K):
        mx = jnp.maximum(mx, x3[:, k, :])
    out_ref[...] = mx


def kernel(xyz, feats, W1, b1, g1, be1, W2, b2, g2, be2):
    idxc = jnp.linspace(0.0, P - 1, M).astype(jnp.int32)
    centers = jnp.take(xyz, idxc, axis=1)

    xyz_t = xyz.transpose(0, 2, 1).reshape(B, 3, NCH, SCH)
    featsT = feats.transpose(0, 2, 1)
    w1t = W1.T
    b1r = b1.reshape(1, C1)

    idx, cproj, ptable = _knn_project(xyz_t, xyz, featsT, centers, w1t, b1r)

    g = _sc_gather(ptable.reshape(B * P, C1), idx.reshape(ROWS))
    g3 = g.reshape(B * M, K, C1)
    cpf = cproj.reshape(B * M, C1)

    nblk = (B * M) // RB
    sums1 = pl.pallas_call(
        _k3_body,
        grid=(nblk,),
        in_specs=[
            pl.BlockSpec((RB, K, C1), lambda i: (i, 0, 0)),
            pl.BlockSpec((RB, C1), lambda i: (i, 0)),
        ],
        out_specs=pl.BlockSpec((8, C1), lambda i: (0, 0)),
        out_shape=jax.ShapeDtypeStruct((8, C1), jnp.float32),
    )(g3, cpf)

    n1 = float(ROWS)
    mean1 = sums1[0] / n1
    var1 = sums1[1] / n1 - mean1 * mean1
    sc1 = g1 / jnp.sqrt(var1 + EPS)
    sh1 = be1 - mean1 * sc1
    w2t = W2.T
    b2r = b2.reshape(1, C2)

    sums2 = pl.pallas_call(
        _k4_body,
        grid=(nblk,),
        in_specs=[
            pl.BlockSpec((RB, K, C1), lambda i: (i, 0, 0)),
            pl.BlockSpec((RB, C1), lambda i: (i, 0)),
            pl.BlockSpec((1, C1), lambda i: (0, 0)),
            pl.BlockSpec((1, C1), lambda i: (0, 0)),
            pl.BlockSpec((C1, C2), lambda i: (0, 0)),
            pl.BlockSpec((1, C2), lambda i: (0, 0)),
        ],
        out_specs=pl.BlockSpec((8, C2), lambda i: (0, 0)),
        out_shape=jax.ShapeDtypeStruct((8, C2), jnp.float32),
    )(g3, cpf, sc1.reshape(1, C1), sh1.reshape(1, C1), w2t, b2r)

    mean2 = sums2[0] / n1
    var2 = sums2[1] / n1 - mean2 * mean2
    sc2 = g2 / jnp.sqrt(var2 + EPS)
    sh2 = be2 - mean2 * sc2

    out2 = pl.pallas_call(
        _k5_body,
        grid=(nblk,),
        in_specs=[
            pl.BlockSpec((RB, K, C1), lambda i: (i, 0, 0)),
            pl.BlockSpec((RB, C1), lambda i: (i, 0)),
            pl.BlockSpec((1, C1), lambda i: (0, 0)),
            pl.BlockSpec((1, C1), lambda i: (0, 0)),
            pl.BlockSpec((C1, C2), lambda i: (0, 0)),
            pl.BlockSpec((1, C2), lambda i: (0, 0)),
            pl.BlockSpec((1, C2), lambda i: (0, 0)),
            pl.BlockSpec((1, C2), lambda i: (0, 0)),
        ],
        out_specs=pl.BlockSpec((RB, C2), lambda i: (i, 0)),
        out_shape=jax.ShapeDtypeStruct((B * M, C2), jnp.float32),
    )(g3, cpf, sc1.reshape(1, C1), sh1.reshape(1, C1), w2t, b2r,
      sc2.reshape(1, C2), sh2.reshape(1, C2))

    out = out2.reshape(B, M, C2).transpose(0, 2, 1)
    return centers, out


# speedup vs baseline: 3.3987x; 1.0734x over previous
import functools

import jax
import jax.numpy as jnp
from jax import lax
from jax.experimental import pallas as pl
from jax.experimental.pallas import tpu as pltpu
from jax.experimental.pallas import tpu_sc as plsc

B, P, C_IN = 4, 8192, 16
M = P // 4
K = 32
C1, C2 = 32, 64
BM = 256
PB = P // (M // BM)
RB = 256
NW = 32
ROWS = B * M * K
CH = 128
EPS = 1e-5


NCH = 64
SCH = 128
DL = 5


def _k1_body(xyzt_ref, xyz_ref, featsT_ref, cen_ref, w1t_ref, b1_ref,
             idx_ref, cproj_ref, ptab_ref):
    xt4 = xyzt_ref[0]
    cen = cen_ref[0]
    pn = jnp.sum(xt4 * xt4, axis=0)
    cn = jnp.sum(cen * cen, axis=1)
    cb = cen.astype(jnp.bfloat16).astype(jnp.float32)
    xb = xt4.astype(jnp.bfloat16).astype(jnp.float32)
    dot = (cb[:, 0][None, :, None] * xb[0][:, None, :]
           + cb[:, 1][None, :, None] * xb[1][:, None, :]
           + cb[:, 2][None, :, None] * xb[2][:, None, :])
    d0 = cn[None, :, None] + pn[:, None, :] - 2.0 * dot

    iota_g = lax.broadcasted_iota(jnp.int32, (NCH, BM, SCH), 0)
    iota_l = lax.broadcasted_iota(jnp.int32, (NCH, BM, SCH), 2)
    gi4 = iota_g * SCH + iota_l
    lane_iota = lax.broadcasted_iota(jnp.int32, (BM, SCH), 1)
    liota = lax.broadcasted_iota(jnp.int32, (DL, BM, SCH), 0)
    kiota = lax.broadcasted_iota(jnp.int32, (BM, K), 1)

    def lev(l, carry):
        dw, v, i = carry
        m = jnp.min(dw, axis=0)
        sel = jnp.where(dw == m[None], iota_g, NCH)
        a = jnp.min(sel, axis=0)
        dw = jnp.where(iota_g == a[None], jnp.inf, dw)
        gi = a * SCH + lane_iota
        v = jnp.where(liota == l, m[None], v)
        i = jnp.where(liota == l, gi[None], i)
        return dw, v, i

    _, v, i = lax.fori_loop(0, DL, lev, (
        d0,
        jnp.full((DL, BM, SCH), jnp.inf, jnp.float32),
        jnp.zeros((DL, BM, SCH), jnp.int32)))

    def step(k, carry):
        w, wi, dc, acc, lastm, lasti = carry
        m = jnp.min(w, axis=1)
        mc = m[:, None]
        ii = jnp.min(jnp.where(w == mc, wi, P), axis=1)
        iic = ii[:, None]
        acc = jnp.where(kiota == k, iic, acc)
        sel = (w == mc) & (wi == iic)
        nv = jnp.full_like(w, jnp.inf)
        ni = jnp.zeros_like(wi)
        for l in range(DL - 1, 0, -1):
            hit = dc == l
            nv = jnp.where(hit, v[l], nv)
            ni = jnp.where(hit, i[l], ni)
        w = jnp.where(sel, nv, w)
        wi = jnp.where(sel, ni, wi)
        dc = dc + sel.astype(jnp.int32)
        return w, wi, dc, acc, m, ii

    _, _, _, acc, lastm, lasti = lax.fori_loop(0, K, step, (
        v[0], i[0], jnp.ones((BM, SCH), jnp.int32),
        jnp.zeros((BM, K), jnp.int32),
        jnp.zeros((BM,), jnp.float32), jnp.zeros((BM,), jnp.int32)))

    lm3 = lastm[None, :, None]
    li3 = lasti[None, :, None]
    lex = (d0 < lm3) | ((d0 == lm3) & (gi4 < li3))
    cnt = jnp.sum(jnp.sum(lex.astype(jnp.int32), axis=0), axis=1)
    bad = jnp.any(cnt != K - 1)

    def _naive():
        def nstep(k, carry):
            dd, acc2 = carry
            m = jnp.min(jnp.min(dd, axis=0), axis=1)[None, :, None]
            ci = jnp.min(jnp.min(jnp.where(dd == m, gi4, P), axis=0), axis=1)
            acc2 = jnp.where(kiota == k, ci[:, None], acc2)
            dd = jnp.where(gi4 == ci[None, :, None], jnp.inf, dd)
            return dd, acc2

        return lax.fori_loop(0, K, nstep,
                             (d0, jnp.zeros((BM, K), jnp.int32)))[1]

    acc = lax.cond(bad, _naive, lambda: acc)
    b = pl.program_id(0)
    idx_ref[0] = acc + b * P

    w1t = w1t_ref[...]
    cproj = (cen[:, 0:1] * w1t[0:1, :]
             + cen[:, 1:2] * w1t[1:2, :]
             + cen[:, 2:3] * w1t[2:3, :]) - b1_ref[...]
    cproj_ref[0] = cproj

    xb = xyz_ref[0]
    fb = featsT_ref[0]
    pt = (xb[:, 0:1] * w1t[0:1, :]
          + xb[:, 1:2] * w1t[1:2, :]
          + xb[:, 2:3] * w1t[2:3, :])
    pt = pt + jnp.dot(fb, w1t[3:, :], preferred_element_type=jnp.float32)
    ptab_ref[0] = pt


def _knn_project(xyz_t, xyz, featsT, centers, w1t, b1r):
    return pl.pallas_call(
        _k1_body,
        grid=(B, M // BM),
        in_specs=[
            pl.BlockSpec((1, 3, NCH, SCH), lambda b, i: (b, 0, 0, 0)),
            pl.BlockSpec((1, PB, 3), lambda b, i: (b, i, 0)),
            pl.BlockSpec((1, PB, C_IN), lambda b, i: (b, i, 0)),
            pl.BlockSpec((1, BM, 3), lambda b, i: (b, i, 0)),
            pl.BlockSpec((3 + C_IN, C1), lambda b, i: (0, 0)),
            pl.BlockSpec((1, C1), lambda b, i: (0, 0)),
        ],
        out_specs=[
            pl.BlockSpec((1, BM, K), lambda b, i: (b, i, 0)),
            pl.BlockSpec((1, BM, C1), lambda b, i: (b, i, 0)),
            pl.BlockSpec((1, PB, C1), lambda b, i: (b, i, 0)),
        ],
        out_shape=[
            jax.ShapeDtypeStruct((B, M, K), jnp.int32),
            jax.ShapeDtypeStruct((B, M, C1), jnp.float32),
            jax.ShapeDtypeStruct((B, P, C1), jnp.float32),
        ],
    )(xyz_t, xyz, featsT, centers, w1t, b1r)


@functools.partial(
    pl.kernel,
    mesh=plsc.VectorSubcoreMesh(core_axis_name="c", subcore_axis_name="s"),
    compiler_params=pltpu.CompilerParams(use_tc_tiling_on_sc=False),
    out_type=jax.ShapeDtypeStruct((ROWS, C1), jnp.float32),
    scratch_types=[
        pltpu.VMEM((CH,), jnp.int32),
        pltpu.VMEM((CH, C1), jnp.float32),
        pltpu.SemaphoreType.DMA,
    ],
)
def _sc_gather(table_hbm, idx_hbm, out_hbm, idx_v, rows_v, sem):
    wid = lax.axis_index("s") * 2 + lax.axis_index("c")
    per_w = ROWS // NW
    base = wid * per_w

    def body(c, carry):
        off = base + c * CH
        pltpu.sync_copy(idx_hbm.at[pl.ds(off, CH)], idx_v)
        pltpu.async_copy(table_hbm.at[idx_v], rows_v, sem).wait()
        pltpu.sync_copy(rows_v, out_hbm.at[pl.ds(off, CH)])
        return carry

    lax.fori_loop(0, per_w // CH, body, 0)


def _k3_body(g_ref, cp_ref, sums_ref):
    h1 = g_ref[...] - cp_ref[...][:, None, :]
    s1 = jnp.sum(jnp.sum(h1, axis=0), axis=0)
    s2 = jnp.sum(jnp.sum(h1 * h1, axis=0), axis=0)

    @pl.when(pl.program_id(0) == 0)
    def _():
        sums_ref[...] = jnp.zeros_like(sums_ref)

    sums_ref[0:1, :] += s1[None, :]
    sums_ref[1:2, :] += s2[None, :]


def _k4_body(g_ref, cp_ref, a1_ref, s1_ref, w2t_ref, b2_ref, sums_ref):
    h1 = g_ref[...] - cp_ref[...][:, None, :]
    x1 = jnp.maximum(h1 * a1_ref[0][None, None, :]
                     + s1_ref[0][None, None, :], 0.0)
    x1f = x1.reshape(RB * K, C1)
    h2 = jnp.dot(x1f, w2t_ref[...], preferred_element_type=jnp.float32)
    h2 = h2 + b2_ref[...]
    s1 = jnp.sum(h2, axis=0)
    s2 = jnp.sum(h2 * h2, axis=0)

    @pl.when(pl.program_id(0) == 0)
    def _():
        sums_ref[...] = jnp.zeros_like(sums_ref)

    sums_ref[0:1, :] += s1[None, :]
    sums_ref[1:2, :] += s2[None, :]


def _k5_body(g_ref, cp_ref, a1_ref, s1_ref, w2t_ref, b2_ref, a2_ref, s2_ref,
             out_ref):
    h1 = g_ref[...] - cp_ref[...][:, None, :]
    x1 = jnp.maximum(h1 * a1_ref[0][None, None, :]
                     + s1_ref[0][None, None, :], 0.0)
    x1f = x1.reshape(RB * K, C1)
    h2 = jnp.dot(x1f, w2t_ref[...], preferred_element_type=jnp.float32)
    h2 = h2 + b2_ref[...]
    x2 = jnp.maximum(h2 * a2_ref[...] + s2_ref[...], 0.0)
    x3 = x2.reshape(RB, K, C2)
    mx = x3[:, 0, :]
    for k in range(1, K):
        mx = jnp.maximum(mx, x3[:, k, :])
    out_ref[...] = mx


def kernel(xyz, feats, W1, b1, g1, be1, W2, b2, g2, be2):
    idxc = jnp.linspace(0.0, P - 1, M).astype(jnp.int32)
    centers = jnp.take(xyz, idxc, axis=1)

    xyz_t = xyz.transpose(0, 2, 1).reshape(B, 3, NCH, SCH)
    featsT = feats.transpose(0, 2, 1)
    w1t = W1.T
    b1r = b1.reshape(1, C1)

    idx, cproj, ptable = _knn_project(xyz_t, xyz, featsT, centers, w1t, b1r)

    g = _sc_gather(ptable.reshape(B * P, C1), idx.reshape(ROWS))
    g3 = g.reshape(B * M, K, C1)
    cpf = cproj.reshape(B * M, C1)

    nblk = (B * M) // RB
    sums1 = pl.pallas_call(
        _k3_body,
        grid=(nblk,),
        in_specs=[
            pl.BlockSpec((RB, K, C1), lambda i: (i, 0, 0)),
            pl.BlockSpec((RB, C1), lambda i: (i, 0)),
        ],
        out_specs=pl.BlockSpec((8, C1), lambda i: (0, 0)),
        out_shape=jax.ShapeDtypeStruct((8, C1), jnp.float32),
    )(g3, cpf)

    n1 = float(ROWS)
    mean1 = sums1[0] / n1
    var1 = sums1[1] / n1 - mean1 * mean1
    sc1 = g1 / jnp.sqrt(var1 + EPS)
    sh1 = be1 - mean1 * sc1
    w2t = W2.T
    b2r = b2.reshape(1, C2)

    sums2 = pl.pallas_call(
        _k4_body,
        grid=(nblk,),
        in_specs=[
            pl.BlockSpec((RB, K, C1), lambda i: (i, 0, 0)),
            pl.BlockSpec((RB, C1), lambda i: (i, 0)),
            pl.BlockSpec((1, C1), lambda i: (0, 0)),
            pl.BlockSpec((1, C1), lambda i: (0, 0)),
            pl.BlockSpec((C1, C2), lambda i: (0, 0)),
            pl.BlockSpec((1, C2), lambda i: (0, 0)),
        ],
        out_specs=pl.BlockSpec((8, C2), lambda i: (0, 0)),
        out_shape=jax.ShapeDtypeStruct((8, C2), jnp.float32),
    )(g3, cpf, sc1.reshape(1, C1), sh1.reshape(1, C1), w2t, b2r)

    mean2 = sums2[0] / n1
    var2 = sums2[1] / n1 - mean2 * mean2
    sc2 = g2 / jnp.sqrt(var2 + EPS)
    sh2 = be2 - mean2 * sc2

    out2 = pl.pallas_call(
        _k5_body,
        grid=(nblk,),
        in_specs=[
            pl.BlockSpec((RB, K, C1), lambda i: (i, 0, 0)),
            pl.BlockSpec((RB, C1), lambda i: (i, 0)),
            pl.BlockSpec((1, C1), lambda i: (0, 0)),
            pl.BlockSpec((1, C1), lambda i: (0, 0)),
            pl.BlockSpec((C1, C2), lambda i: (0, 0)),
            pl.BlockSpec((1, C2), lambda i: (0, 0)),
            pl.BlockSpec((1, C2), lambda i: (0, 0)),
            pl.BlockSpec((1, C2), lambda i: (0, 0)),
        ],
        out_specs=pl.BlockSpec((RB, C2), lambda i: (i, 0)),
        out_shape=jax.ShapeDtypeStruct((B * M, C2), jnp.float32),
    )(g3, cpf, sc1.reshape(1, C1), sh1.reshape(1, C1), w2t, b2r,
      sc2.reshape(1, C2), sh2.reshape(1, C2))

    out = out2.reshape(B, M, C2).transpose(0, 2, 1)
    return centers, out
